# baseline (speedup 1.0000x reference)
import functools

import jax
import jax.numpy as jnp
from jax import lax
from jax.experimental import pallas as pl
from jax.experimental.pallas import tpu as pltpu
from jax.experimental.pallas import tpu_sc as plsc

_NC = 2
_NS = 16
_NL = 16
_C = 64
_NB = 2
_SEGC = 8


def _enc_body(xn_ref, xh_ref, wn_ref, wh_ref, on_ref, oh_ref):
    on_ref[...] = jnp.dot(xn_ref[...], wn_ref[...],
                          preferred_element_type=jnp.float32)
    oh_ref[...] = jnp.dot(xh_ref[...], wh_ref[...],
                          preferred_element_type=jnp.float32)


def _encode(x_node_feat, x_he_feat, W_node, W_he):
    N, DF = x_node_feat.shape
    DH = W_node.shape[1]
    BR = 1000
    return pl.pallas_call(
        _enc_body,
        grid=(N // BR,),
        in_specs=[
            pl.BlockSpec((BR, DF), lambda i: (i, 0)),
            pl.BlockSpec((BR, DF), lambda i: (i, 0)),
            pl.BlockSpec((DF, DH), lambda i: (0, 0)),
            pl.BlockSpec((DF, DH), lambda i: (0, 0)),
        ],
        out_specs=[
            pl.BlockSpec((BR, DH), lambda i: (i, 0)),
            pl.BlockSpec((BR, DH), lambda i: (i, 0)),
        ],
        out_shape=[
            jax.ShapeDtypeStruct((N, DH), jnp.float32),
            jax.ShapeDtypeStruct((N, DH), jnp.float32),
        ],
    )(x_node_feat, x_he_feat, W_node, W_he)


def _fin_body(n_valid, p0_ref, p1_ref, u0_ref, u1_ref, bd_ref,
              keep_ref, cnt_ref):
    thr = (jnp.log(-jnp.log(u1_ref[...])) - jnp.log(-jnp.log(u0_ref[...]))
           - bd_ref[...])
    s = p0_ref[...] + p1_ref[...]
    R, Ccol = keep_ref.shape
    flat = (lax.broadcasted_iota(jnp.int32, (R, Ccol), 0) * Ccol
            + lax.broadcasted_iota(jnp.int32, (R, Ccol), 1))
    keep = jnp.where((flat < n_valid) & (s > thr), 1.0, 0.0)
    keep = keep.astype(jnp.float32)
    keep_ref[...] = keep
    cnt_ref[...] = jnp.sum(keep).reshape(1, 1)


def _finalize(p0, p1, u0, u1, bd_row, n_valid):
    R, Ccol = u0.shape
    return pl.pallas_call(
        functools.partial(_fin_body, n_valid),
        out_shape=[
            jax.ShapeDtypeStruct((R, Ccol), jnp.float32),
            jax.ShapeDtypeStruct((1, 1), jnp.float32),
        ],
    )(p0, p1, u0, u1, bd_row)


def _rne_bf16(x):
    b = plsc.bitcast(x, jnp.uint32)
    lsb = (b >> jnp.uint32(16)) & jnp.uint32(1)
    r = (b + jnp.uint32(0x7FFF) + lsb) & jnp.uint32(0xFFFF0000)
    return plsc.bitcast(r, jnp.float32)


def _decode_sc(xn_sp, xh_sp, wd_sp, src_3d, dst_3d):
    _, NNODE, DHH = xn_sp.shape
    _, nchunk, _ = src_3d.shape
    e_pad = _NS * nchunk * _C
    per_w = nchunk * _C
    nseg = nchunk // _SEGC
    segw = _SEGC * _C
    mesh = plsc.VectorSubcoreMesh(core_axis_name="c", subcore_axis_name="s")

    @functools.partial(
        pl.kernel,
        mesh=mesh,
        out_type=jax.ShapeDtypeStruct((_NC, e_pad), jnp.float32),
        scratch_types=[
            pltpu.VMEM((_NC, _SEGC, _C), jnp.int32),
            pltpu.VMEM((_NC, _SEGC, _C), jnp.int32),
            pltpu.VMEM((2, segw), jnp.float32),
            pltpu.VMEM((_NB, _C, DHH), jnp.float32),
            pltpu.VMEM((_NB, _C, DHH), jnp.float32),
            pltpu.VMEM((DHH,), jnp.float32),
            pltpu.VMEM_SHARED((NNODE, DHH), jnp.float32),
            pltpu.VMEM_SHARED((NNODE, DHH), jnp.float32),
        ] + [pltpu.SemaphoreType.DMA] * 8,
        compiler_params=pltpu.CompilerParams(needs_layout_passes=False,
                                             use_tc_tiling_on_sc=False),
    )
    def k(xn_hbm, xh_hbm, wd_hbm, src_hbm, dst_hbm, psum_hbm,
          srcseg, dstseg, dotseg, av, bv, wdv, spxn, spxh,
          gA0, gB0, gA1, gB1, sS, sD, st0, st1):
        sid = lax.axis_index("s")
        cid = lax.axis_index("c")
        base_w = sid * per_w
        iota16 = lax.iota(jnp.int32, _NL)
        gsem = ((gA0, gB0), (gA1, gB1))
        stsem = (st0, st1)

        pltpu.sync_copy(wd_hbm.at[cid], wdv)

        @pl.when(sid == 0)
        def _():
            pltpu.sync_copy(xn_hbm.at[cid], spxn)
            pltpu.sync_copy(xh_hbm.at[cid], spxh)

        plsc.subcore_barrier()

        def stage_seg(sg, slot):
            pltpu.async_copy(src_hbm.at[sid].at[pl.ds(sg * _SEGC, _SEGC)],
                             srcseg.at[slot], sS)
            pltpu.async_copy(dst_hbm.at[sid].at[pl.ds(sg * _SEGC, _SEGC)],
                             dstseg.at[slot], sD)

        def wait_seg(sg, slot):
            pltpu.make_async_copy(
                src_hbm.at[sid].at[pl.ds(sg * _SEGC, _SEGC)],
                srcseg.at[slot], sS).wait()
            pltpu.make_async_copy(
                dst_hbm.at[sid].at[pl.ds(sg * _SEGC, _SEGC)],
                dstseg.at[slot], sD).wait()

        def start_g(segslot, j, slot):
            pltpu.async_copy(spxn.at[srcseg.at[segslot, j]],
                             av.at[slot], gsem[slot][0])
            pltpu.async_copy(spxh.at[dstseg.at[segslot, j]],
                             bv.at[slot], gsem[slot][1])

        def wait_g(segslot, j, slot):
            pltpu.make_async_copy(spxn.at[srcseg.at[segslot, j]],
                                  av.at[slot], gsem[slot][0]).wait()
            pltpu.make_async_copy(spxh.at[dstseg.at[segslot, j]],
                                  bv.at[slot], gsem[slot][1]).wait()

        def psum_dst(sg):
            return psum_hbm.at[cid].at[pl.ds(base_w + sg * segw, segw)]

        def compute(j, slot, segslot):
            a2d = av.at[slot]
            b2d = bv.at[slot]

            def group_body(g, _):
                rows = g * _NL + iota16
                accs = [jnp.zeros((_NL,), jnp.float32) for _ in range(4)]
                for kk in range(DHH):
                    c = (kk + iota16) & (DHH - 1)
                    p = _rne_bf16(plsc.load_gather(a2d, [rows, c])
                                  * plsc.load_gather(b2d, [rows, c]))
                    w = plsc.load_gather(wdv, [c])
                    accs[kk % 4] = accs[kk % 4] + p * w
                acc = (accs[0] + accs[1]) + (accs[2] + accs[3])
                dotseg[segslot, pl.ds(j * _C + g * _NL, _NL)] = acc
                return 0

            lax.fori_loop(0, _C // _NL, group_body, 0)

        pltpu.sync_copy(src_hbm.at[sid].at[pl.ds(0, _SEGC)], srcseg.at[0])
        pltpu.sync_copy(dst_hbm.at[sid].at[pl.ds(0, _SEGC)], dstseg.at[0])
        stage_seg(1, 1)
        start_g(0, 0, 0)
        start_g(0, 1, 1)

        def seg_body(sg, _):
            cur = sg & 1
            nxt = 1 - cur

            @pl.when((sg >= 2) & (cur == 0))
            def _():
                pltpu.make_async_copy(dotseg.at[0], psum_dst(sg - 2),
                                      stsem[0]).wait()

            @pl.when((sg >= 2) & (cur == 1))
            def _():
                pltpu.make_async_copy(dotseg.at[1], psum_dst(sg - 2),
                                      stsem[1]).wait()

            @pl.when(sg + 1 < nseg)
            def _():
                stage_seg(sg + 1, nxt)

            npair = _SEGC // _NB

            def pair_body(jp, _):
                for b in range(_NB):
                    j = jp * _NB + b
                    wait_g(cur, j, b)
                    compute(j, b, cur)

                    @pl.when(jp < npair - 1)
                    def _():
                        start_g(cur, j + _NB, b)

                    if b == 0:
                        @pl.when((jp == npair - 1) & (sg + 1 < nseg))
                        def _():
                            wait_seg(sg + 1, nxt)
                            start_g(nxt, 0, b)
                    else:
                        @pl.when((jp == npair - 1) & (sg + 1 < nseg))
                        def _():
                            start_g(nxt, b, b)
                return 0

            lax.fori_loop(0, npair, pair_body, 0)

            @pl.when(cur == 0)
            def _():
                pltpu.async_copy(dotseg.at[0], psum_dst(sg), stsem[0])

            @pl.when(cur == 1)
            def _():
                pltpu.async_copy(dotseg.at[1], psum_dst(sg), stsem[1])

            return 0

        lax.fori_loop(0, nseg, seg_body, 0)

        pltpu.make_async_copy(dotseg.at[(nseg - 2) & 1], psum_dst(nseg - 2),
                              stsem[(nseg - 2) & 1]).wait()
        pltpu.make_async_copy(dotseg.at[(nseg - 1) & 1], psum_dst(nseg - 1),
                              stsem[(nseg - 1) & 1]).wait()

    return k(xn_sp, xh_sp, wd_sp, src_3d, dst_3d)


def kernel(x_node_feat, x_he_feat, W_node, W_he, W_dec, b_dec, edge_index,
           num_ori_edge, gumbel_u):
    n_ori = gumbel_u.shape[0]
    n_edges = edge_index.shape[1]
    DH = W_node.shape[1]
    DHH = DH // _NC
    blk = _NS * _C * _SEGC
    e_pad = ((n_ori + blk - 1) // blk) * blk
    per_w = e_pad // _NS
    nchunk = per_w // _C

    wdb = (W_dec[:, 1].astype(jnp.bfloat16).astype(jnp.float32)
           - W_dec[:, 0].astype(jnp.bfloat16).astype(jnp.float32))
    wd_sp = wdb.reshape(_NC, DHH)
    bd = b_dec[1] - b_dec[0]
    gcol = 128
    bd_row = jnp.full((1, gcol), bd, jnp.float32)

    zero_dep = jnp.asarray(num_ori_edge, dtype=edge_index.dtype) - n_ori
    src_p = jnp.pad(edge_index[0, :n_ori] + zero_dep,
                    (0, e_pad - n_ori)).astype(jnp.int32)
    dst_p = jnp.pad(edge_index[1, :n_ori] + zero_dep,
                    (0, e_pad - n_ori)).astype(jnp.int32)
    src_3d = src_p.reshape(_NS, nchunk, _C)
    dst_3d = dst_p.reshape(_NS, nchunk, _C)

    gup = jnp.pad(gumbel_u, ((0, e_pad - n_ori), (0, 0)), constant_values=0.5)
    R = e_pad // gcol
    u0 = gup[:, 0].reshape(R, gcol)
    u1 = gup[:, 1].reshape(R, gcol)

    xn, xh = _encode(x_node_feat, x_he_feat, W_node, W_he)
    N = xn.shape[0]
    xn_sp = xn.reshape(N, _NC, DHH).transpose(1, 0, 2)
    xh_sp = xh.reshape(N, _NC, DHH).transpose(1, 0, 2)

    psum = _decode_sc(xn_sp, xh_sp, wd_sp, src_3d, dst_3d)

    p0 = psum[0].reshape(R, gcol)
    p1 = psum[1].reshape(R, gcol)
    keep2d, cnt = _finalize(p0, p1, u0, u1, bd_row, n_ori)

    keep = keep2d.reshape(e_pad)[:n_ori]
    deg = 1.0 - cnt[0, 0] / jnp.float32(n_ori)
    full = jnp.concatenate(
        [keep, jnp.ones((n_edges - n_ori,), jnp.float32)], axis=0)
    return (full, deg)


# speedup vs baseline: 1.1516x; 1.1516x over previous
import functools

import jax
import jax.numpy as jnp
from jax import lax
from jax.experimental import pallas as pl
from jax.experimental.pallas import tpu as pltpu
from jax.experimental.pallas import tpu_sc as plsc

_NC = 2
_NS = 16
_NL = 16
_NW = _NC * _NS
_KT = 4
_C = 1024
_NB = 2


def _enc_body(xn_ref, xh_ref, wn_ref, wh_ref, on_ref, oh_ref):
    on_ref[...] = jnp.dot(xn_ref[...], wn_ref[...],
                          preferred_element_type=jnp.float32)
    oh_ref[...] = jnp.dot(xh_ref[...], wh_ref[...],
                          preferred_element_type=jnp.float32)


def _encode(x_node_feat, x_he_feat, W_node, W_he):
    N, DF = x_node_feat.shape
    DH = W_node.shape[1]
    BR = 1000
    return pl.pallas_call(
        _enc_body,
        grid=(N // BR,),
        in_specs=[
            pl.BlockSpec((BR, DF), lambda i: (i, 0)),
            pl.BlockSpec((BR, DF), lambda i: (i, 0)),
            pl.BlockSpec((DF, DH), lambda i: (0, 0)),
            pl.BlockSpec((DF, DH), lambda i: (0, 0)),
        ],
        out_specs=[
            pl.BlockSpec((BR, DH), lambda i: (i, 0)),
            pl.BlockSpec((BR, DH), lambda i: (i, 0)),
        ],
        out_shape=[
            jax.ShapeDtypeStruct((N, DH), jnp.float32),
            jax.ShapeDtypeStruct((N, DH), jnp.float32),
        ],
    )(x_node_feat, x_he_feat, W_node, W_he)


_BR = 8


def _fin_body(n_valid, ps_ref, u0_ref, u1_ref, bd_ref, keep_ref, cnt_ref):
    i = pl.program_id(0)
    s = jnp.sum(ps_ref[...], axis=0)
    thr = (jnp.log(-jnp.log(u1_ref[...])) - jnp.log(-jnp.log(u0_ref[...]))
           - bd_ref[...])
    Ccol = keep_ref.shape[1]
    flat = ((i * _BR + lax.broadcasted_iota(jnp.int32, (_BR, Ccol), 0)) * Ccol
            + lax.broadcasted_iota(jnp.int32, (_BR, Ccol), 1))
    keep = jnp.where((flat < n_valid) & (s > thr), 1.0, 0.0)
    keep = keep.astype(jnp.float32)
    keep_ref[...] = keep

    @pl.when(i == 0)
    def _():
        cnt_ref[...] = jnp.zeros((1, 1), jnp.float32)

    cnt_ref[...] += jnp.sum(keep).reshape(1, 1)


def _finalize(ps3, u0, u1, bd_row, n_valid):
    R, Ccol = u0.shape
    return pl.pallas_call(
        functools.partial(_fin_body, n_valid),
        grid=(R // _BR,),
        in_specs=[
            pl.BlockSpec((_NW, _BR, Ccol), lambda i: (0, i, 0)),
            pl.BlockSpec((_BR, Ccol), lambda i: (i, 0)),
            pl.BlockSpec((_BR, Ccol), lambda i: (i, 0)),
            pl.BlockSpec((1, Ccol), lambda i: (0, 0)),
        ],
        out_specs=[
            pl.BlockSpec((_BR, Ccol), lambda i: (i, 0)),
            pl.BlockSpec((1, 1), lambda i: (0, 0)),
        ],
        out_shape=[
            jax.ShapeDtypeStruct((R, Ccol), jnp.float32),
            jax.ShapeDtypeStruct((1, 1), jnp.float32),
        ],
    )(ps3, u0, u1, bd_row)


def _rne_bf16(x):
    b = plsc.bitcast(x, jnp.uint32)
    lsb = (b >> jnp.uint32(16)) & jnp.uint32(1)
    r = (b + jnp.uint32(0x7FFF) + lsb) & jnp.uint32(0xFFFF0000)
    return plsc.bitcast(r, jnp.float32)


def _decode_sc(xt, ht, wd_pad, src_p, dst_p):
    _, _, NNODE = xt.shape
    e_pad = src_p.shape[0]
    nchunk = e_pad // _C
    ngroup = _C // _NL
    mesh = plsc.VectorSubcoreMesh(core_axis_name="c", subcore_axis_name="s")

    @functools.partial(
        pl.kernel,
        mesh=mesh,
        out_type=jax.ShapeDtypeStruct((_NW, e_pad), jnp.float32),
        scratch_types=[
            pltpu.VMEM((_KT, NNODE), jnp.float32),
            pltpu.VMEM((_KT, NNODE), jnp.float32),
            pltpu.VMEM((_NB, _C), jnp.int32),
            pltpu.VMEM((_NB, _C), jnp.int32),
            pltpu.VMEM((_NB, _C), jnp.float32),
            pltpu.VMEM((_NW * _KT + _NL,), jnp.float32),
        ] + [pltpu.SemaphoreType.DMA] * 6,
        compiler_params=pltpu.CompilerParams(needs_layout_passes=False,
                                             use_tc_tiling_on_sc=False),
    )
    def k(xt_hbm, ht_hbm, wd_hbm, src_hbm, dst_hbm, psum_hbm,
          ta, tb, srcr, dstr, psr, wdv, sS0, sD0, sS1, sD1, sP0, sP1):
        sid = lax.axis_index("s")
        cid = lax.axis_index("c")
        wid = sid * _NC + cid
        iota16 = lax.iota(jnp.int32, _NL)
        ssem = (sS0, sS1)
        dsem = (sD0, sD1)
        psem = (sP0, sP1)

        pltpu.sync_copy(xt_hbm.at[wid], ta)
        pltpu.sync_copy(ht_hbm.at[wid], tb)
        pltpu.sync_copy(wd_hbm, wdv)
        wsl = wdv[pl.ds(wid * _KT, _NL)]
        wks = [wsl[j] for j in range(_KT)]
        kvecs = [jnp.zeros((_NL,), jnp.int32) + j for j in range(_KT)]

        def stage(ci, b):
            pltpu.async_copy(src_hbm.at[pl.ds(ci * _C, _C)], srcr.at[b],
                             ssem[b])
            pltpu.async_copy(dst_hbm.at[pl.ds(ci * _C, _C)], dstr.at[b],
                             dsem[b])

        def wait_stage(ci, b):
            pltpu.make_async_copy(src_hbm.at[pl.ds(ci * _C, _C)], srcr.at[b],
                                  ssem[b]).wait()
            pltpu.make_async_copy(dst_hbm.at[pl.ds(ci * _C, _C)], dstr.at[b],
                                  dsem[b]).wait()

        def ps_dst(ci):
            return psum_hbm.at[wid].at[pl.ds(ci * _C, _C)]

        def store_ps(ci, b):
            pltpu.async_copy(psr.at[b], ps_dst(ci), psem[b])

        def wait_ps(ci, b):
            pltpu.make_async_copy(psr.at[b], ps_dst(ci), psem[b]).wait()

        def compute(ci, b):
            def group_body(g, _):
                off = g * _NL
                srcv = srcr[b, pl.ds(off, _NL)]
                dstv = dstr[b, pl.ds(off, _NL)]
                acc = jnp.zeros((_NL,), jnp.float32)
                a1 = jnp.zeros((_NL,), jnp.float32)
                for j in range(_KT):
                    ga = plsc.load_gather(ta, [kvecs[j], srcv])
                    gb = plsc.load_gather(tb, [kvecs[j], dstv])
                    p = _rne_bf16(ga * gb)
                    if j % 2 == 0:
                        acc = acc + p * wks[j]
                    else:
                        a1 = a1 + p * wks[j]
                psr[b, pl.ds(off, _NL)] = acc + a1
                return 0

            lax.fori_loop(0, ngroup, group_body, 0)

        stage(0, 0)
        stage(1, 1)

        def pair_body(jp, _):
            for b in range(_NB):
                ci = jp * _NB + b
                wait_stage(ci, b)

                @pl.when(ci >= _NB)
                def _():
                    wait_ps(ci - _NB, b)

                compute(ci, b)
                store_ps(ci, b)

                @pl.when(ci + _NB < nchunk)
                def _():
                    stage(ci + _NB, b)
            return 0

        lax.fori_loop(0, nchunk // _NB, pair_body, 0)
        wait_ps(nchunk - 2, 0)
        wait_ps(nchunk - 1, 1)

    return k(xt, ht, wd_pad, src_p, dst_p)


def kernel(x_node_feat, x_he_feat, W_node, W_he, W_dec, b_dec, edge_index,
           num_ori_edge, gumbel_u):
    n_ori = gumbel_u.shape[0]
    n_edges = edge_index.shape[1]
    DH = W_node.shape[1]
    blk = _C * _NB
    e_pad = ((n_ori + blk - 1) // blk) * blk

    wdb = (W_dec[:, 1].astype(jnp.bfloat16).astype(jnp.float32)
           - W_dec[:, 0].astype(jnp.bfloat16).astype(jnp.float32))
    wd_pad = jnp.pad(wdb, (0, _NL))
    bd = b_dec[1] - b_dec[0]
    gcol = 128
    bd_row = jnp.full((1, gcol), bd, jnp.float32)

    zero_dep = jnp.asarray(num_ori_edge, dtype=edge_index.dtype) - n_ori
    src_p = jnp.pad(edge_index[0, :n_ori] + zero_dep,
                    (0, e_pad - n_ori)).astype(jnp.int32)
    dst_p = jnp.pad(edge_index[1, :n_ori] + zero_dep,
                    (0, e_pad - n_ori)).astype(jnp.int32)

    gup = jnp.pad(gumbel_u, ((0, e_pad - n_ori), (0, 0)), constant_values=0.5)
    R = e_pad // gcol
    u0 = gup[:, 0].reshape(R, gcol)
    u1 = gup[:, 1].reshape(R, gcol)

    xn, xh = _encode(x_node_feat, x_he_feat, W_node, W_he)
    N = xn.shape[0]
    xt = xn.T.reshape(_NW, _KT, N)
    ht = xh.T.reshape(_NW, _KT, N)

    psum = _decode_sc(xt, ht, wd_pad, src_p, dst_p)

    ps3 = psum.reshape(_NW, R, gcol)
    keep2d, cnt = _finalize(ps3, u0, u1, bd_row, n_ori)

    keep = keep2d.reshape(e_pad)[:n_ori]
    deg = 1.0 - cnt[0, 0] / jnp.float32(n_ori)
    full = jnp.concatenate(
        [keep, jnp.ones((n_edges - n_ori,), jnp.float32)], axis=0)
    return (full, deg)
